# Initial kernel scaffold; baseline (speedup 1.0000x reference)
import jax
import jax.numpy as jnp
from jax.experimental import pallas as pl


def kernel(embeddings, edge_index, W1, b1, W2, b2, R1, Rb1, R2, Rb2):
    raise NotImplementedError("write your pallas kernel here")


# speedup vs baseline: 6.2149x; 6.2149x over previous
import functools

import jax
import jax.numpy as jnp
from jax import lax
from jax.experimental import pallas as pl
from jax.experimental.pallas import tpu as pltpu
from jax.experimental.pallas import tpu_sc as plsc

N = 10000
E = 320000
D = 128
H = 32

NC = 2
NS = 16
NW = NC * NS
EPW = E // NW
BLK = 80
NBLK = EPW // BLK
NCHUNK = BLK // 16
GSUB = 80
NGS = BLK // GSUB
NPAD = NW * 320
NR = 320
BIG = 1 << 30


def _tc_tables(emb_ref, wcat_ref, g_ref):
    g_ref[...] = jnp.dot(emb_ref[...].astype(jnp.bfloat16),
                         wcat_ref[...].astype(jnp.bfloat16),
                         preferred_element_type=jnp.float32)


def _sc_edges(src_hbm, dst_hbm, g_hbm, cst_hbm,
              scores_hbm, wall_hbm, call_hbm,
              src_v, dst_v, g1r, g2r, st, sc_v, win_v, cmp_v, cst_v, sem):
    wid = lax.axis_index("s") * NC + lax.axis_index("c")
    iota = lax.iota(jnp.int32, 16)

    pltpu.sync_copy(cst_hbm, cst_v)

    def init_body(i, carry):
        win_v[pl.ds(i * 16, 16)] = jnp.full((16,), -1, jnp.int32)
        cmp_v[pl.ds(i * 16, 16)] = jnp.full((16,), 0, jnp.int32)
        return carry
    lax.fori_loop(0, NPAD // 16, init_body, 0)

    b2v = cst_v[pl.ds(2 * H, 16)]
    b1c = [cst_v[pl.ds(0, 16)], cst_v[pl.ds(16, 16)]]
    w2c = [cst_v[pl.ds(H, 16)], cst_v[pl.ds(H + 16, 16)]]
    b1e = [b1c[j // 16][j % 16] for j in range(H)]
    w2e = [w2c[j // 16][j % 16] for j in range(H)]

    def block_body(b, carry):
        base = pl.multiple_of(wid * EPW + b * BLK, 8)
        pltpu.sync_copy(src_hbm.at[pl.ds(base, BLK)], src_v)
        pltpu.sync_copy(dst_hbm.at[pl.ds(base, BLK)], dst_v)
        descs = []
        for g in range(NGS):
            sl = pl.ds(g * GSUB, GSUB)
            descs.append(pltpu.async_copy(g_hbm.at[src_v.at[sl]], g1r.at[sl], sem))
            descs.append(pltpu.async_copy(g_hbm.at[dst_v.at[sl]], g2r.at[sl], sem))
        for dsc in descs:
            dsc.wait()


        def chunk_body(c, carry):
            eoff = c * 16
            idxbase = iota * 16
            for l in range(16):
                e = eoff + l
                for half in range(2):
                    srow = (g1r[e, pl.ds(half * 16, 16)]
                            + g2r[e, pl.ds(H + half * 16, 16)])
                    plsc.store_scatter(st, [idxbase + (half * 256 + l)], srow)
            acc = b2v
            for j in range(H):
                featj = st[pl.ds(j * 16, 16)]
                hj = jnp.maximum(featj + b1e[j], 0.0)
                u = plsc.bitcast(hj, jnp.int32)
                u = lax.bitwise_and(
                    u + 0x7FFF + lax.bitwise_and(
                        lax.shift_right_logical(u, 16), 1),
                    jnp.int32(-65536))
                acc = acc + plsc.bitcast(u, jnp.float32) * w2e[j]
            score = 1.0 / (1.0 + jnp.exp(-acc))
            sc_v[pl.ds(eoff, 16)] = score
            valid = acc > 0.0

            srcc = src_v[pl.ds(eoff, 16)]
            dstc = dst_v[pl.ds(eoff, 16)]
            for phase in range(2):
                nid = srcc if phase == 0 else dstc
                other = dstc if phase == 0 else srcc
                key = jnp.where(valid, nid * 16 + iota, BIG)
                ksort = lax.sort(key)
                sidx = lax.shift_right_arithmetic(ksort, 4)
                lorig = lax.bitwise_and(ksort, 15)
                nxt = ksort.at[jnp.minimum(iota + 1, 15)].get(
                    mode="promise_in_bounds")
                keep = jnp.logical_or(
                    sidx != lax.shift_right_arithmetic(nxt, 4), iota == 15)
                m = jnp.logical_and(keep, ksort < BIG)
                sidx = jnp.where(m, sidx, 0)
                prio = base + eoff + lorig + phase * E
                old = plsc.load_gather(win_v, [sidx], mask=m)
                wins = jnp.logical_and(m, prio > old)
                plsc.store_scatter(win_v, [sidx], prio, mask=wins)
                othv = other.at[lorig].get(mode="promise_in_bounds")
                plsc.store_scatter(cmp_v, [sidx], othv, mask=wins)
            return carry
        lax.fori_loop(0, NCHUNK, chunk_body, 0)
        pltpu.sync_copy(sc_v, scores_hbm.at[pl.ds(base, BLK)])
        return carry
    lax.fori_loop(0, NBLK, block_body, 0)

    wbase = pl.multiple_of(wid * NPAD, 8)
    pltpu.sync_copy(win_v, wall_hbm.at[pl.ds(wbase, NPAD)])
    pltpu.sync_copy(cmp_v, call_hbm.at[pl.ds(wbase, NPAD)])


def _sc_nodes(wall_hbm, call_hbm, g_hbm, rb_hbm,
              hr_hbm, wm_hbm,
              wtmp, ctmp, wbest_v, cbest_v, av, bv, g1r, g2r, hrv, rb_v, sem):
    wid = lax.axis_index("s") * NC + lax.axis_index("c")
    lo = pl.multiple_of(wid * NR, 8)
    iota = lax.iota(jnp.int32, 16)

    pltpu.sync_copy(rb_hbm, rb_v)

    def init_body(c, carry):
        off = pl.ds(c * 16, 16)
        wbest_v[off] = jnp.full((16,), -1, jnp.int32)
        cbest_v[off] = jnp.full((16,), 0, jnp.int32)
        return carry
    lax.fori_loop(0, NR // 16, init_body, 0)

    def t_body(t, carry):
        soff = pl.multiple_of(t * NPAD + lo, 8)
        pltpu.sync_copy(wall_hbm.at[pl.ds(soff, NR)], wtmp)
        pltpu.sync_copy(call_hbm.at[pl.ds(soff, NR)], ctmp)

        def c_body(c, carry2):
            off = pl.ds(c * 16, 16)
            wb = wbest_v[off]
            wt = wtmp[off]
            better = wt > wb
            wbest_v[off] = jnp.where(better, wt, wb)
            cbest_v[off] = jnp.where(better, ctmp[off], cbest_v[off])
            return carry2
        lax.fori_loop(0, NR // 16, c_body, 0)
        return carry
    lax.fori_loop(0, NW, t_body, 0)

    def ab_body(c, carry):
        off = pl.ds(c * 16, 16)
        wb = wbest_v[off]
        cb = cbest_v[off]
        has = wb >= 0
        isdst = wb >= E
        v = lo + c * 16 + iota
        av[off] = jnp.where(has, jnp.where(isdst, cb, v), 0)
        bv[off] = jnp.where(has, jnp.where(isdst, v, cb), 0)
        return carry
    lax.fori_loop(0, NR // 16, ab_body, 0)

    for g in range(NR // GSUB):
        sl = pl.ds(g * GSUB, GSUB)
        d1 = pltpu.async_copy(g_hbm.at[av.at[sl]], g1r, sem)
        d2 = pltpu.async_copy(g_hbm.at[bv.at[sl]], g2r, sem)
        d1.wait()
        d2.wait()

        def hr_body(r, carry):
            for half in range(2):
                off = pl.ds(half * 16, 16)
                x = (g1r[r, pl.ds(2 * H + half * 16, 16)]
                     + g2r[r, pl.ds(3 * H + half * 16, 16)] + rb_v[off])
                hrv[g * GSUB + r, off] = jnp.maximum(x, 0.0)
            return carry
        lax.fori_loop(0, GSUB, hr_body, 0)

    pltpu.sync_copy(hrv, hr_hbm.at[pl.ds(lo, NR)])
    pltpu.sync_copy(wbest_v, wm_hbm.at[pl.ds(lo, NR)])


def _tc_finish(emb_ref, hr_ref, wm_ref, r2t_ref, rb2_ref, out_ref):
    res = jnp.dot(hr_ref[...].astype(jnp.bfloat16),
                  r2t_ref[...].astype(jnp.bfloat16),
                  preferred_element_type=jnp.float32) + rb2_ref[...]
    cond = wm_ref[...] >= 0
    emb = emb_ref[...]
    out_ref[...] = jnp.where(cond, (emb + res) * 0.5, emb)


def kernel(embeddings, edge_index, W1, b1, W2, b2, R1, Rb1, R2, Rb2):
    f32 = jnp.float32
    src = edge_index[0]
    dst = edge_index[1]
    wcat = jnp.concatenate(
        [W1[:, :D].T, W1[:, D:].T, R1[:, :D].T, R1[:, D:].T], axis=1)
    uw = jax.lax.bitcast_convert_type(W2[0], jnp.int32)
    uw = (uw + 0x7FFF + ((uw >> 16) & 1)) & jnp.int32(-65536)
    w2r = jax.lax.bitcast_convert_type(uw, f32)
    cst = jnp.concatenate([b1, w2r, jnp.full((16,), b2[0], f32)])

    gtab = pl.pallas_call(
        _tc_tables,
        out_shape=jax.ShapeDtypeStruct((N, 4 * H), f32),
    )(embeddings, wcat)

    mesh = plsc.VectorSubcoreMesh(
        core_axis_name="c", subcore_axis_name="s",
        num_cores=NC, num_subcores=NS)

    sc_params = pltpu.CompilerParams(needs_layout_passes=False)
    edges_k = pl.kernel(
        _sc_edges,
        compiler_params=sc_params,
        out_type=[
            jax.ShapeDtypeStruct((E,), f32),
            jax.ShapeDtypeStruct((NW * NPAD,), jnp.int32),
            jax.ShapeDtypeStruct((NW * NPAD,), jnp.int32),
        ],
        mesh=mesh,
        scratch_types=[
            pltpu.VMEM((BLK,), jnp.int32),
            pltpu.VMEM((BLK,), jnp.int32),
            pltpu.VMEM((BLK, 4 * H), f32),
            pltpu.VMEM((BLK, 4 * H), f32),
            pltpu.VMEM((H * 16,), f32),
            pltpu.VMEM((BLK,), f32),
            pltpu.VMEM((NPAD,), jnp.int32),
            pltpu.VMEM((NPAD,), jnp.int32),
            pltpu.VMEM((2 * H + 16,), f32),
            pltpu.SemaphoreType.DMA,
        ],
    )
    scores, wall, callarr = edges_k(src, dst, gtab, cst)

    nodes_k = pl.kernel(
        _sc_nodes,
        compiler_params=sc_params,
        out_type=[
            jax.ShapeDtypeStruct((NPAD, H), f32),
            jax.ShapeDtypeStruct((NPAD,), jnp.int32),
        ],
        mesh=mesh,
        scratch_types=[
            pltpu.VMEM((NR,), jnp.int32),
            pltpu.VMEM((NR,), jnp.int32),
            pltpu.VMEM((NR,), jnp.int32),
            pltpu.VMEM((NR,), jnp.int32),
            pltpu.VMEM((NR,), jnp.int32),
            pltpu.VMEM((NR,), jnp.int32),
            pltpu.VMEM((GSUB, 4 * H), f32),
            pltpu.VMEM((GSUB, 4 * H), f32),
            pltpu.VMEM((NR, H), f32),
            pltpu.VMEM((H,), f32),
            pltpu.SemaphoreType.DMA,
        ],
    )
    hr, wm = nodes_k(wall, callarr, gtab, Rb1)

    resolved = pl.pallas_call(
        _tc_finish,
        out_shape=jax.ShapeDtypeStruct((N, D), f32),
    )(embeddings, hr[:N], wm[:N].reshape(N, 1), R2.T, Rb2.reshape(1, D))

    return resolved, scores
